# Initial kernel scaffold; baseline (speedup 1.0000x reference)
import jax
import jax.numpy as jnp
from jax.experimental import pallas as pl


def kernel(h, pos, vel, acc, crowd, mask, idex, hist_feature, W1, b1, W2, b2, W3, b3, ln_g, ln_b):
    raise NotImplementedError("write your pallas kernel here")


# speedup vs baseline: 10.7227x; 10.7227x over previous
import functools

import jax
import jax.numpy as jnp
import numpy as np
from jax.experimental import pallas as pl

B, N, K, H = 32, 256, 32, 8
D = 128
CROWD = 5
CH = 64
NCH = N // CH
E = CH * K


def _edge_kernel(h_ref, pos_ref, vel_ref, hist_ref,
                 hc_ref, posc_ref, velc_ref, accc_ref, crowdc_ref, histc_ref,
                 mask_ref, idxE_ref, ideE_ref,
                 W1h_ref, W1s_ref, b1_ref, W2_ref, b2_ref,
                 W3h_ref, W3a_ref, W3c_ref, b3_ref, lng_ref, lnb_ref,
                 wt2_ref, S48_ref, w8_ref, out_ref):
    f32 = jnp.float32
    h = h_ref[0]
    pos = pos_ref[0]
    vel = vel_ref[0]
    hist = hist_ref[0]

    hW1 = jnp.dot(h, W1h_ref[...], preferred_element_type=f32) + b1_ref[...]
    histA = hist * wt2_ref[...]
    q = jnp.dot(histA * hist, S48_ref[...], preferred_element_type=f32)
    pv = jnp.concatenate([pos, vel], axis=-1)
    T2 = jnp.concatenate([hist, q], axis=-1)

    h_c = hc_ref[0]
    pos_c = posc_ref[0]
    vel_c = velc_ref[0]
    acc_c = accc_ref[0]
    crowd_c = crowdc_ref[0]
    hist_c = histc_ref[0]

    mu = jnp.mean(crowd_c, axis=-1, keepdims=True)
    var = jnp.mean((crowd_c - mu) ** 2, axis=-1, keepdims=True)
    crowd1 = (crowd_c - mu) * jax.lax.rsqrt(var + 1e-5) * lng_ref[...] + lnb_ref[...]
    node_base = (jnp.dot(h_c, W3h_ref[...], preferred_element_type=f32)
                 + jnp.dot(crowd1, W3c_ref[...], preferred_element_type=f32)
                 + b3_ref[...])
    pm = jnp.concatenate([vel_c, acc_c], axis=-1)
    ped_norm = jnp.sqrt(jnp.sum(pm * pm, -1, keepdims=True))
    cm = crowd_c[:, :4]
    crowd_norm = jnp.sqrt(jnp.sum(cm * cm, -1, keepdims=True))
    dotpc = jnp.sum(pm * cm, -1, keepdims=True)
    csim = (dotpc / (ped_norm * crowd_norm + 1e-6) + 1.0) * 0.5
    histA_c = hist_c * wt2_ref[...]
    q_c = jnp.dot(histA_c * hist_c, S48_ref[...], preferred_element_type=f32)

    m = mask_ref[0]
    idx = idxE_ref[0, 0]
    ide = ideE_ref[0, 0]
    jiota = jax.lax.broadcasted_iota(jnp.int32, (E, N), 1)
    OH1 = (jiota == idx).astype(f32)
    OH2 = (jiota == ide).astype(f32)
    g1 = jnp.dot(OH1, hW1, preferred_element_type=f32)
    gpv = jnp.dot(OH1, pv, preferred_element_type=f32)
    g2 = jnp.dot(OH2, T2, preferred_element_type=f32)

    pvj = gpv.reshape(CH, K, 4)
    histj = g2[:, :48].reshape(CH, K, 48)
    qj = g2[:, 48:56].reshape(CH, K, 8)

    rel = pvj[:, :, 0:2] - pos_c.reshape(CH, 1, 2)
    dist = jnp.sqrt(jnp.sum(rel * rel, -1, keepdims=True)) + 1e-6
    dv = vel_c.reshape(CH, 1, 2) - pvj[:, :, 2:4]
    rspeed = jnp.sqrt(jnp.sum(dv * dv, -1, keepdims=True))

    cross = jnp.dot((histA_c.reshape(CH, 1, 48) * histj).reshape(E, 48),
                    S48_ref[...], preferred_element_type=f32).reshape(CH, K, 8)
    d2 = jnp.maximum(q_c.reshape(CH, 1, 8) + qj - 2.0 * cross, 0.0)
    simt = jnp.exp(-jnp.sqrt(d2))
    w8 = w8_ref[...].reshape(1, 1, H)
    hsim = jnp.sum(simt * w8, -1, keepdims=True) * 0.1
    csim_e = jnp.broadcast_to(csim.reshape(CH, 1, 1), (CH, K, 1))

    scal = jnp.concatenate(
        [rel, dist, csim_e, hsim, rspeed, jnp.zeros((CH, K, 2), f32)], axis=-1)
    zs = jnp.dot(scal.reshape(E, 8), W1s_ref[...], preferred_element_type=f32)
    e1 = jnp.maximum(g1 + zs, 0.0).reshape(CH, K, D)
    s = jnp.sum(e1 * m.reshape(CH, K, 1), axis=1)
    msum = jnp.sum(m, axis=1, keepdims=True)
    aggn = jnp.dot(s, W2_ref[...], preferred_element_type=f32) + msum * b2_ref[...]
    agg = aggn / (msum + 1e-6)

    o = node_base + jnp.dot(agg, W3a_ref[...], preferred_element_type=f32)
    out_ref[0] = jnp.maximum(o, 0.0)


@jax.jit
def kernel(h, pos, vel, acc, crowd, mask, idex, hist_feature,
           W1, b1, W2, b2, W3, b3, ln_g, ln_b):
    f32 = jnp.float32
    hist = hist_feature.reshape(B, N, H * 6)
    W1h = W1[:D]
    W1s = jnp.concatenate([W1[D:D + 6], jnp.zeros((2, D), f32)], axis=0)
    W3h = W3[:D]
    W3a = W3[D:2 * D]
    W3c = W3[2 * D:2 * D + CROWD]
    wt = np.array([0.1, 0.1, 1.0, 1.0, 0.5, 0.5], np.float32)
    wt2 = jnp.asarray(np.tile(wt * wt, H).reshape(1, H * 6))
    S48 = jnp.asarray(np.kron(np.eye(H, dtype=np.float32),
                              np.ones((6, 1), np.float32)))
    wts = 0.8 ** np.arange(H - 1, -1, -1, dtype=np.float32)
    w8 = jnp.asarray((wts / (wts.sum() + 1e-6)).reshape(1, H))
    idxE = (idex.astype(f32) * mask).astype(jnp.int32).reshape(B, NCH, E, 1)
    ideE = idex.reshape(B, NCH, E, 1)

    grid = (B, NCH)
    bcast = lambda shape: pl.BlockSpec(shape, lambda b, c: (0,) * len(shape))
    perb = lambda shape: pl.BlockSpec((1,) + shape, lambda b, c: (b, 0, 0))
    chunk = lambda last: pl.BlockSpec((1, CH, last), lambda b, c: (b, c, 0))
    out = pl.pallas_call(
        _edge_kernel,
        grid=grid,
        in_specs=[
            perb((N, D)),
            perb((N, 2)), perb((N, 2)),
            perb((N, H * 6)),
            chunk(D), chunk(2), chunk(2), chunk(2),
            chunk(CROWD), chunk(H * 6),
            chunk(K),
            pl.BlockSpec((1, 1, E, 1), lambda b, c: (b, c, 0, 0)),
            pl.BlockSpec((1, 1, E, 1), lambda b, c: (b, c, 0, 0)),
            bcast((D, D)), bcast((8, D)), bcast((1, D)),
            bcast((D, D)), bcast((1, D)),
            bcast((D, D)), bcast((D, D)), bcast((CROWD, D)), bcast((1, D)),
            bcast((1, CROWD)), bcast((1, CROWD)),
            bcast((1, H * 6)), bcast((H * 6, H)), bcast((1, H)),
        ],
        out_specs=pl.BlockSpec((1, CH, D), lambda b, c: (b, c, 0)),
        out_shape=jax.ShapeDtypeStruct((B, N, D), f32),
    )(h, pos, vel, hist,
      h, pos, vel, acc, crowd, hist,
      mask, idxE, ideE,
      W1h, W1s, b1.reshape(1, D), W2, b2.reshape(1, D),
      W3h, W3a, W3c, b3.reshape(1, D),
      ln_g.reshape(1, CROWD), ln_b.reshape(1, CROWD), wt2, S48, w8)
    return out


# speedup vs baseline: 26.4170x; 2.4636x over previous
import functools

import jax
import jax.numpy as jnp
import numpy as np
from jax.experimental import pallas as pl

B, N, K, H = 32, 256, 32, 8
D = 128
CROWD = 5
CH = 64
NCH = N // CH
E = CH * K


def _edge_kernel(hT_ref, posT_ref, velT_ref, accT_ref, crowdT_ref, histT_ref,
                 maskE_ref, idxE_ref, ideE_ref,
                 W1hT_ref, W1s6T_ref, b1_ref, W2T_ref, b2_ref,
                 W3hT_ref, W3aT_ref, W3cT_ref, b3_ref, lng_ref, lnb_ref,
                 wt2c_ref, S48T_ref, w8_ref, out_ref):
    f32 = jnp.float32
    hT = hT_ref[0]
    posT = posT_ref[0]
    velT = velT_ref[0]
    accT = accT_ref[0]
    crowdT = crowdT_ref[0]
    histT = histT_ref[0]
    c = pl.program_id(1)
    r0 = c * CH

    hW1T = jnp.dot(W1hT_ref[...], hT, preferred_element_type=f32) + b1_ref[...]
    histAT = histT * wt2c_ref[...]
    qT = jnp.dot(S48T_ref[...], histAT * histT,
                 preferred_element_type=f32)
    pvT = jnp.concatenate([posT, velT], axis=0)
    T2T = jnp.concatenate([histT, qT], axis=0)

    pmT = jnp.concatenate([velT, accT], axis=0)
    ped_norm = jnp.sqrt(jnp.sum(pmT * pmT, 0, keepdims=True))
    cmT = crowdT[0:4]
    crowd_norm = jnp.sqrt(jnp.sum(cmT * cmT, 0, keepdims=True))
    dotpc = jnp.sum(pmT * cmT, 0, keepdims=True)
    csimT = (dotpc / (ped_norm * crowd_norm + 1e-6) + 1.0) * 0.5

    mu = jnp.mean(crowdT, 0, keepdims=True)
    var = jnp.mean((crowdT - mu) ** 2, 0, keepdims=True)
    crowd1T = ((crowdT - mu) * jax.lax.rsqrt(var + 1e-5) * lng_ref[...]
               + lnb_ref[...])
    node_baseT = (jnp.dot(W3hT_ref[...], hT, preferred_element_type=f32)
                  + jnp.dot(W3cT_ref[...], crowd1T, preferred_element_type=f32)
                  + b3_ref[...])

    SelT = (jax.lax.broadcasted_iota(jnp.int32, (N, CH), 0) ==
            jax.lax.broadcasted_iota(jnp.int32, (N, CH), 1) + r0).astype(f32)
    Xp = (jax.lax.broadcasted_iota(jnp.int32, (CH, E), 0) ==
          jax.lax.broadcasted_iota(jnp.int32, (CH, E), 1) // K).astype(f32)
    XpT = (jax.lax.broadcasted_iota(jnp.int32, (E, CH), 0) // K ==
           jax.lax.broadcasted_iota(jnp.int32, (E, CH), 1)).astype(f32)

    TBLq = jnp.concatenate([qT, histAT, pvT, csimT], axis=0)
    QcT = jnp.dot(TBLq, SelT, preferred_element_type=f32)
    QeT = jnp.dot(QcT, Xp, preferred_element_type=f32)
    qiT = QeT[0:8]
    histAiT = QeT[8:56]
    posiT = QeT[56:58]
    veliT = QeT[58:60]
    csimiT = QeT[60:61]

    m = maskE_ref[0, 0]
    idx = idxE_ref[0, 0]
    ide = ideE_ref[0, 0]
    jiota = jax.lax.broadcasted_iota(jnp.int32, (N, E), 0)
    OH1T = (jiota == idx).astype(f32)
    OH2T = (jiota == ide).astype(f32)
    G1 = jnp.dot(jnp.concatenate([hW1T, pvT], axis=0), OH1T,
                 preferred_element_type=f32)
    g2 = jnp.dot(T2T, OH2T, preferred_element_type=f32)
    g1T = G1[0:D]
    pvjT = G1[D:D + 4]
    histjT = g2[0:48]
    qjT = g2[48:56]

    relT = pvjT[0:2] - posiT
    distT = jnp.sqrt(jnp.sum(relT * relT, 0, keepdims=True)) + 1e-6
    dvT = veliT - pvjT[2:4]
    rspeedT = jnp.sqrt(jnp.sum(dvT * dvT, 0, keepdims=True))
    crossT = jnp.dot(S48T_ref[...], histAiT * histjT,
                     preferred_element_type=f32)
    d2 = jnp.maximum(qiT + qjT - 2.0 * crossT, 0.0)
    simtT = jnp.exp(-jnp.sqrt(d2))
    hsimT = jnp.dot(w8_ref[...], simtT, preferred_element_type=f32) * 0.1
    scalT = jnp.concatenate([relT, distT, csimiT, hsimT, rspeedT], axis=0)

    zsT = jnp.dot(W1s6T_ref[...], scalT, preferred_element_type=f32)
    e1T = jnp.maximum(g1T + zsT, 0.0) * m
    sT = jnp.dot(e1T, XpT, preferred_element_type=f32)
    msum = jnp.dot(m, XpT, preferred_element_type=f32)
    aggT = ((jnp.dot(W2T_ref[...], sT, preferred_element_type=f32)
             + b2_ref[...] * msum) / (msum + 1e-6))

    nbT = jnp.dot(node_baseT, SelT, preferred_element_type=f32)
    oT = jnp.maximum(nbT + jnp.dot(W3aT_ref[...], aggT,
                                   preferred_element_type=f32), 0.0)
    out_ref[0] = oT.T


@jax.jit
def kernel(h, pos, vel, acc, crowd, mask, idex, hist_feature,
           W1, b1, W2, b2, W3, b3, ln_g, ln_b):
    f32 = jnp.float32
    hT = jnp.swapaxes(h, 1, 2)
    posT = jnp.swapaxes(pos, 1, 2)
    velT = jnp.swapaxes(vel, 1, 2)
    accT = jnp.swapaxes(acc, 1, 2)
    crowdT = jnp.swapaxes(crowd, 1, 2)
    histT = jnp.swapaxes(hist_feature.reshape(B, N, H * 6), 1, 2)

    W1hT = W1[:D].T
    W1s6T = W1[D:D + 6].T
    W2T = W2.T
    W3hT = W3[:D].T
    W3aT = W3[D:2 * D].T
    W3cT = W3[2 * D:2 * D + CROWD].T
    wt = np.array([0.1, 0.1, 1.0, 1.0, 0.5, 0.5], np.float32)
    wt2c = jnp.asarray(np.tile(wt * wt, H).reshape(H * 6, 1))
    S48T = jnp.asarray(np.kron(np.eye(H, dtype=np.float32),
                               np.ones((1, 6), np.float32)))
    wts = 0.8 ** np.arange(H - 1, -1, -1, dtype=np.float32)
    w8 = jnp.asarray((wts / (wts.sum() + 1e-6)).reshape(1, H))

    maskE = mask.reshape(B, NCH, 1, E)
    idxE = (idex.astype(f32) * mask).astype(jnp.int32).reshape(B, NCH, 1, E)
    ideE = idex.reshape(B, NCH, 1, E)

    grid = (B, NCH)
    bcast = lambda shape: pl.BlockSpec(shape, lambda b, c: (0,) * len(shape))
    perb = lambda shape: pl.BlockSpec((1,) + shape, lambda b, c: (b, 0, 0))
    edge = pl.BlockSpec((1, 1, 1, E), lambda b, c: (b, c, 0, 0))
    out = pl.pallas_call(
        _edge_kernel,
        grid=grid,
        in_specs=[
            perb((D, N)),
            perb((2, N)), perb((2, N)), perb((2, N)),
            perb((CROWD, N)),
            perb((H * 6, N)),
            edge, edge, edge,
            bcast((D, D)), bcast((D, 6)), bcast((D, 1)),
            bcast((D, D)), bcast((D, 1)),
            bcast((D, D)), bcast((D, D)), bcast((D, CROWD)), bcast((D, 1)),
            bcast((CROWD, 1)), bcast((CROWD, 1)),
            bcast((H * 6, 1)), bcast((H, H * 6)), bcast((1, H)),
        ],
        out_specs=pl.BlockSpec((1, CH, D), lambda b, c: (b, c, 0)),
        out_shape=jax.ShapeDtypeStruct((B, N, D), f32),
    )(hT, posT, velT, accT, crowdT, histT,
      maskE, idxE, ideE,
      W1hT, W1s6T, b1.reshape(D, 1), W2T, b2.reshape(D, 1),
      W3hT, W3aT, W3cT, b3.reshape(D, 1),
      ln_g.reshape(CROWD, 1), ln_b.reshape(CROWD, 1), wt2c, S48T, w8)
    return out
